# Initial kernel scaffold; baseline (speedup 1.0000x reference)
import functools

import jax
import jax.numpy as jnp
from jax import lax
from jax.experimental import pallas as pl
from jax.experimental.pallas import tpu as pltpu
from jax.experimental.pallas import tpu_sc as plsc

_NUM_CORES = 2
_NUM_SUBCORES = 16
_NUM_WORKERS = _NUM_CORES * _NUM_SUBCORES
_CHUNK = 128


@functools.lru_cache(maxsize=None)
def _make_gather(V, D, B):
    n_per_w = B // _NUM_WORKERS
    n_chunks = n_per_w // _CHUNK
    mesh = plsc.VectorSubcoreMesh(core_axis_name="c", subcore_axis_name="s")

    @functools.partial(
        pl.kernel,
        mesh=mesh,
        out_type=jax.ShapeDtypeStruct((B, D), jnp.float32),
        scratch_types=[
            pltpu.VMEM((n_chunks, _CHUNK), jnp.int32),
            pltpu.VMEM((_CHUNK, D), jnp.float32),
            pltpu.SemaphoreType.DMA,
        ],
    )
    def gather_kernel(idx_hbm, table_hbm, out_hbm, idx_v, rows_v, sem):
        wid = lax.axis_index("s") * _NUM_CORES + lax.axis_index("c")
        pltpu.sync_copy(idx_hbm.at[wid], idx_v)

        def body(j, carry):
            pltpu.async_copy(table_hbm.at[idx_v.at[j]], rows_v, sem).wait()
            base = wid * n_per_w + j * _CHUNK
            pltpu.sync_copy(rows_v, out_hbm.at[pl.ds(base, _CHUNK)])
            return carry

        lax.fori_loop(0, n_chunks, body, 0)

    return gather_kernel


def kernel(token_ids, weight):
    B0, S = token_ids.shape
    V, D = weight.shape
    B = B0 * S
    idx = token_ids.reshape(
        _NUM_WORKERS, B // (_NUM_WORKERS * _CHUNK), _CHUNK
    ).astype(jnp.int32)
    out = _make_gather(V, D, B)(idx, weight)
    return out.reshape(B0, S, D)


# speedup vs baseline: 1.6854x; 1.6854x over previous
import functools

import jax
import jax.numpy as jnp
from jax import lax
from jax.experimental import pallas as pl
from jax.experimental.pallas import tpu as pltpu
from jax.experimental.pallas import tpu_sc as plsc

_NUM_CORES = 2
_NUM_SUBCORES = 16
_NUM_WORKERS = _NUM_CORES * _NUM_SUBCORES
_CHUNK = 128


@functools.lru_cache(maxsize=None)
def _make_gather(V, D, B):
    n_per_w = B // _NUM_WORKERS
    n_chunks = n_per_w // _CHUNK
    mesh = plsc.VectorSubcoreMesh(core_axis_name="c", subcore_axis_name="s")

    @functools.partial(
        pl.kernel,
        mesh=mesh,
        out_type=jax.ShapeDtypeStruct((B, D), jnp.float32),
        scratch_types=[
            pltpu.VMEM((n_chunks, _CHUNK), jnp.int32),
            pltpu.VMEM((_CHUNK, D), jnp.float32),
            pltpu.SemaphoreType.DMA,
        ],
        compiler_params=pltpu.CompilerParams(use_tc_tiling_on_sc=False),
    )
    def gather_kernel(idx_hbm, table_hbm, out_hbm, idx_v, rows_v, sem):
        wid = lax.axis_index("s") * _NUM_CORES + lax.axis_index("c")
        pltpu.sync_copy(idx_hbm.at[wid], idx_v)

        def body(j, carry):
            pltpu.async_copy(table_hbm.at[idx_v.at[j]], rows_v, sem).wait()
            base = wid * n_per_w + j * _CHUNK
            pltpu.sync_copy(rows_v, out_hbm.at[pl.ds(base, _CHUNK)])
            return carry

        lax.fori_loop(0, n_chunks, body, 0)

    return gather_kernel


def kernel(token_ids, weight):
    B0, S = token_ids.shape
    V, D = weight.shape
    B = B0 * S
    idx = token_ids.reshape(
        _NUM_WORKERS, B // (_NUM_WORKERS * _CHUNK), _CHUNK
    ).astype(jnp.int32)
    out = _make_gather(V, D, B)(idx, weight)
    return out.reshape(B0, S, D)


# speedup vs baseline: 1.8627x; 1.1052x over previous
import functools

import jax
import jax.numpy as jnp
from jax import lax
from jax.experimental import pallas as pl
from jax.experimental.pallas import tpu as pltpu
from jax.experimental.pallas import tpu_sc as plsc

_NUM_CORES = 2
_NUM_SUBCORES = 16
_NUM_WORKERS = _NUM_CORES * _NUM_SUBCORES
_CHUNK = 128


_K = 4


@functools.lru_cache(maxsize=None)
def _make_gather(V, D, B):
    n_per_w = B // _NUM_WORKERS
    n_chunks = n_per_w // _CHUNK
    n_groups = n_chunks // _K
    assert n_groups % 2 == 0 and n_groups >= 4
    mesh = plsc.VectorSubcoreMesh(core_axis_name="c", subcore_axis_name="s")

    @functools.partial(
        pl.kernel,
        mesh=mesh,
        out_type=jax.ShapeDtypeStruct((B, D), jnp.float32),
        scratch_types=[
            pltpu.VMEM((n_chunks, _CHUNK), jnp.int32),
            pltpu.VMEM((_K, _CHUNK, D), jnp.float32),
            pltpu.VMEM((_K, _CHUNK, D), jnp.float32),
            pltpu.SemaphoreType.DMA,
            pltpu.SemaphoreType.DMA,
            pltpu.SemaphoreType.DMA,
            pltpu.SemaphoreType.DMA,
        ],
        compiler_params=pltpu.CompilerParams(use_tc_tiling_on_sc=False),
    )
    def gather_kernel(idx_hbm, table_hbm, out_hbm, idx_v,
                      rows_a, rows_b, gs_a, gs_b, os_a, os_b):
        wid = lax.axis_index("s") * _NUM_CORES + lax.axis_index("c")
        out_base = wid * n_per_w
        pltpu.sync_copy(idx_hbm.at[wid], idx_v)

        def fire_gathers(g, rows, sem):
            for b in range(_K):
                pltpu.async_copy(table_hbm.at[idx_v.at[g * _K + b]],
                                 rows.at[b], sem)

        def drain_gathers(g, rows, sem):
            for b in range(_K):
                pltpu.make_async_copy(table_hbm.at[idx_v.at[g * _K + b]],
                                      rows.at[b], sem).wait()

        def fire_scatters(g, rows, sem):
            for b in range(_K):
                dst = out_hbm.at[pl.ds(out_base + (g * _K + b) * _CHUNK, _CHUNK)]
                pltpu.async_copy(rows.at[b], dst, sem)

        def drain_scatters(g, rows, sem):
            for b in range(_K):
                dst = out_hbm.at[pl.ds(out_base + (g * _K + b) * _CHUNK, _CHUNK)]
                pltpu.make_async_copy(rows.at[b], dst, sem).wait()

        fire_gathers(0, rows_a, gs_a)

        def body(t, carry):
            ga, gb = 2 * t, 2 * t + 1
            fire_gathers(gb, rows_b, gs_b)
            drain_gathers(ga, rows_a, gs_a)
            fire_scatters(ga, rows_a, os_a)
            drain_gathers(gb, rows_b, gs_b)
            fire_scatters(gb, rows_b, os_b)
            drain_scatters(ga, rows_a, os_a)
            fire_gathers(ga + 2, rows_a, gs_a)
            drain_scatters(gb, rows_b, os_b)
            return carry

        lax.fori_loop(0, n_groups // 2 - 1, body, 0)

        g_last = n_groups - 2
        fire_gathers(g_last + 1, rows_b, gs_b)
        drain_gathers(g_last, rows_a, gs_a)
        fire_scatters(g_last, rows_a, os_a)
        drain_gathers(g_last + 1, rows_b, gs_b)
        fire_scatters(g_last + 1, rows_b, os_b)
        drain_scatters(g_last, rows_a, os_a)
        drain_scatters(g_last + 1, rows_b, os_b)

    return gather_kernel


def kernel(token_ids, weight):
    B0, S = token_ids.shape
    V, D = weight.shape
    B = B0 * S
    idx = token_ids.reshape(
        _NUM_WORKERS, B // (_NUM_WORKERS * _CHUNK), _CHUNK
    ).astype(jnp.int32)
    out = _make_gather(V, D, B)(idx, weight)
    return out.reshape(B0, S, D)
